# initial kernel scaffold (unmeasured)
import jax
import jax.numpy as jnp
from jax import lax
from jax.experimental import pallas as pl
from jax.experimental.pallas import tpu as pltpu


def kernel(
    x,
):
    def body(*refs):
        pass

    out_shape = jax.ShapeDtypeStruct(..., jnp.float32)
    return pl.pallas_call(body, out_shape=out_shape)(...)


# baseline (device time: 433188 ns/iter reference)
import jax
import jax.numpy as jnp
from jax import lax
from jax.experimental import pallas as pl
from jax.experimental.pallas import tpu as pltpu

N_CHUNKS = 8


def kernel(x):
    m, n = x.shape
    rows = m // N_CHUNKS

    def body(x_hbm, out_hbm, recv_hbm, vx, vr, vo,
             send_sem, recv_sem, cp_sems):
        my_x = lax.axis_index("x")
        my_y = lax.axis_index("y")
        my_z = lax.axis_index("z")
        partner = (1 - my_x, my_y, my_z)

        barrier_sem = pltpu.get_barrier_semaphore()
        pl.semaphore_signal(
            barrier_sem, inc=1,
            device_id=partner, device_id_type=pl.DeviceIdType.MESH,
        )
        pl.semaphore_wait(barrier_sem, 1)

        rdma = pltpu.make_async_remote_copy(
            src_ref=x_hbm,
            dst_ref=recv_hbm,
            send_sem=send_sem,
            recv_sem=recv_sem,
            device_id=partner,
            device_id_type=pl.DeviceIdType.MESH,
        )
        rdma.start()
        rdma.wait()

        for c in range(N_CHUNKS):
            sl = pl.ds(c * rows, rows)
            cp_x = pltpu.make_async_copy(x_hbm.at[sl], vx, cp_sems.at[0])
            cp_r = pltpu.make_async_copy(recv_hbm.at[sl], vr, cp_sems.at[1])
            cp_x.start()
            cp_r.start()
            cp_x.wait()
            cp_r.wait()
            vo[:, :] = vx[:, :] + vr[:, :]
            cp_o = pltpu.make_async_copy(vo, out_hbm.at[sl], cp_sems.at[2])
            cp_o.start()
            cp_o.wait()

    out, _recv = pl.pallas_call(
        body,
        out_shape=[
            jax.ShapeDtypeStruct((m, n), x.dtype),
            jax.ShapeDtypeStruct((m, n), x.dtype),
        ],
        in_specs=[pl.BlockSpec(memory_space=pltpu.MemorySpace.HBM)],
        out_specs=[
            pl.BlockSpec(memory_space=pltpu.MemorySpace.HBM),
            pl.BlockSpec(memory_space=pltpu.MemorySpace.HBM),
        ],
        scratch_shapes=[
            pltpu.VMEM((rows, n), x.dtype),
            pltpu.VMEM((rows, n), x.dtype),
            pltpu.VMEM((rows, n), x.dtype),
            pltpu.SemaphoreType.DMA,
            pltpu.SemaphoreType.DMA,
            pltpu.SemaphoreType.DMA((3,)),
        ],
        compiler_params=pltpu.CompilerParams(collective_id=0),
    )(x)
    return out


# device time: 223686 ns/iter; 1.9366x vs baseline; 1.9366x over previous
import jax
import jax.numpy as jnp
from jax import lax
from jax.experimental import pallas as pl
from jax.experimental.pallas import tpu as pltpu

N_CHUNKS = 16


def kernel(x):
    m, n = x.shape
    half = m // 2
    rc = half // N_CHUNKS

    def body(x_hbm, out_hbm, recv_hbm, vx, vr, vs,
             x_send, x_recv, y_send, y_recv, cpx_sem, cpr_sem, cpo_sem):
        my_x = lax.axis_index("x")
        my_y = lax.axis_index("y")
        my_z = lax.axis_index("z")
        x_partner = (1 - my_x, my_y, my_z)
        y_partner = (my_x, my_y ^ 1, my_z)

        my_half = my_y % 2
        half_off = my_half * half

        barrier_sem = pltpu.get_barrier_semaphore()
        for nbr in (x_partner, y_partner):
            pl.semaphore_signal(
                barrier_sem, inc=1,
                device_id=nbr, device_id_type=pl.DeviceIdType.MESH,
            )
        pl.semaphore_wait(barrier_sem, 2)

        x_rdmas = []
        for c in range(N_CHUNKS):
            rdma = pltpu.make_async_remote_copy(
                src_ref=x_hbm.at[pl.ds(half_off + c * rc, rc)],
                dst_ref=recv_hbm.at[pl.ds(c * rc, rc)],
                send_sem=x_send.at[c],
                recv_sem=x_recv.at[c],
                device_id=x_partner,
                device_id_type=pl.DeviceIdType.MESH,
            )
            rdma.start()
            x_rdmas.append(rdma)

        y_rdmas = []
        for c in range(N_CHUNKS):
            slot = c % 2
            if c >= 2:
                y_rdmas[c - 2].wait_send()
            rows = pl.ds(half_off + c * rc, rc)
            cp_x = pltpu.make_async_copy(x_hbm.at[rows], vx.at[slot], cpx_sem.at[slot])
            cp_x.start()
            x_rdmas[c].wait_recv()
            cp_r = pltpu.make_async_copy(
                recv_hbm.at[pl.ds(c * rc, rc)], vr.at[slot], cpr_sem.at[slot]
            )
            cp_r.start()
            cp_x.wait()
            cp_r.wait()
            vs[slot] = vx[slot] + vr[slot]
            cp_o = pltpu.make_async_copy(vs.at[slot], out_hbm.at[rows], cpo_sem.at[slot])
            cp_o.start()
            y_rdma = pltpu.make_async_remote_copy(
                src_ref=vs.at[slot],
                dst_ref=out_hbm.at[rows],
                send_sem=y_send.at[c],
                recv_sem=y_recv.at[c],
                device_id=y_partner,
                device_id_type=pl.DeviceIdType.MESH,
            )
            y_rdma.start()
            y_rdmas.append(y_rdma)
            cp_o.wait()

        y_rdmas[N_CHUNKS - 2].wait_send()
        y_rdmas[N_CHUNKS - 1].wait_send()
        for c in range(N_CHUNKS):
            y_rdmas[c].wait_recv()
            x_rdmas[c].wait_send()

    out, _recv = pl.pallas_call(
        body,
        out_shape=[
            jax.ShapeDtypeStruct((m, n), x.dtype),
            jax.ShapeDtypeStruct((half, n), x.dtype),
        ],
        in_specs=[pl.BlockSpec(memory_space=pltpu.MemorySpace.HBM)],
        out_specs=[
            pl.BlockSpec(memory_space=pltpu.MemorySpace.HBM),
            pl.BlockSpec(memory_space=pltpu.MemorySpace.HBM),
        ],
        scratch_shapes=[
            pltpu.VMEM((2, rc, n), x.dtype),
            pltpu.VMEM((2, rc, n), x.dtype),
            pltpu.VMEM((2, rc, n), x.dtype),
            pltpu.SemaphoreType.DMA((N_CHUNKS,)),
            pltpu.SemaphoreType.DMA((N_CHUNKS,)),
            pltpu.SemaphoreType.DMA((N_CHUNKS,)),
            pltpu.SemaphoreType.DMA((N_CHUNKS,)),
            pltpu.SemaphoreType.DMA((2,)),
            pltpu.SemaphoreType.DMA((2,)),
            pltpu.SemaphoreType.DMA((2,)),
        ],
        compiler_params=pltpu.CompilerParams(collective_id=0),
    )(x)
    return out


# device time: 179663 ns/iter; 2.4111x vs baseline; 1.2450x over previous
import jax
import jax.numpy as jnp
from jax import lax
from jax.experimental import pallas as pl
from jax.experimental.pallas import tpu as pltpu

K = 8


def kernel(x):
    m, n = x.shape
    Q = m // 4
    rc = Q // K
    H = K // 2

    def body(x_hbm, out_hbm, recv_hbm, vx, vr, vs,
             x_send, x_recv, yd_send, yd_recv, zd_send, zd_recv,
             yr_send, yr_recv, zr_send, zr_recv,
             cpx_sem, cpr_sem, cpo_sem):
        my_x = lax.axis_index("x")
        my_y = lax.axis_index("y")
        my_z = lax.axis_index("z")
        p = my_y % 2
        r = my_z % 2
        x_partner = (1 - my_x, my_y, my_z)
        y_partner = (my_x, my_y ^ 1, my_z)
        z_partner = (my_x, my_y, my_z ^ 1)

        q_me = 2 * p + r
        q_yp = 2 * (1 - p) + r
        q_zp = 2 * p + (1 - r)

        barrier_sem = pltpu.get_barrier_semaphore()
        for nbr in (x_partner, y_partner, z_partner):
            pl.semaphore_signal(
                barrier_sem, inc=1,
                device_id=nbr, device_id_type=pl.DeviceIdType.MESH,
            )
        pl.semaphore_wait(barrier_sem, 3)

        x_rdmas = []
        for c in range(K):
            rdma = pltpu.make_async_remote_copy(
                src_ref=x_hbm.at[pl.ds(q_me * Q + c * rc, rc)],
                dst_ref=recv_hbm.at[pl.ds(c * rc, rc)],
                send_sem=x_send.at[c],
                recv_sem=x_recv.at[c],
                device_id=x_partner,
                device_id_type=pl.DeviceIdType.MESH,
            )
            rdma.start()
            x_rdmas.append(rdma)

        yd_rdmas = []
        zd_rdmas = []
        for c in range(K):
            slot = c % 2
            if c >= 2:
                yd_rdmas[c - 2].wait_send()
                zd_rdmas[c - 2].wait_send()
            rows = pl.ds(q_me * Q + c * rc, rc)
            cp_x = pltpu.make_async_copy(x_hbm.at[rows], vx.at[slot], cpx_sem.at[slot])
            cp_x.start()
            x_rdmas[c].wait_recv()
            cp_r = pltpu.make_async_copy(
                recv_hbm.at[pl.ds(c * rc, rc)], vr.at[slot], cpr_sem.at[slot]
            )
            cp_r.start()
            cp_x.wait()
            cp_r.wait()
            vs[slot] = vx[slot] + vr[slot]
            cp_o = pltpu.make_async_copy(vs.at[slot], out_hbm.at[rows], cpo_sem.at[slot])
            cp_o.start()
            yd = pltpu.make_async_remote_copy(
                src_ref=vs.at[slot],
                dst_ref=out_hbm.at[rows],
                send_sem=yd_send.at[c],
                recv_sem=yd_recv.at[c],
                device_id=y_partner,
                device_id_type=pl.DeviceIdType.MESH,
            )
            yd.start()
            yd_rdmas.append(yd)
            zd = pltpu.make_async_remote_copy(
                src_ref=vs.at[slot],
                dst_ref=out_hbm.at[rows],
                send_sem=zd_send.at[c],
                recv_sem=zd_recv.at[c],
                device_id=z_partner,
                device_id_type=pl.DeviceIdType.MESH,
            )
            zd.start()
            zd_rdmas.append(zd)
            cp_o.wait()

        yr_rdmas = []
        zr_rdmas = []
        for c in range(K):
            yd_rdmas[c].wait_recv()
            if c >= H:
                rows = pl.ds(q_yp * Q + c * rc, rc)
                zr = pltpu.make_async_remote_copy(
                    src_ref=out_hbm.at[rows],
                    dst_ref=out_hbm.at[rows],
                    send_sem=zr_send.at[c - H],
                    recv_sem=zr_recv.at[c - H],
                    device_id=z_partner,
                    device_id_type=pl.DeviceIdType.MESH,
                )
                zr.start()
                zr_rdmas.append(zr)
            zd_rdmas[c].wait_recv()
            if c < H:
                rows = pl.ds(q_zp * Q + c * rc, rc)
                yr = pltpu.make_async_remote_copy(
                    src_ref=out_hbm.at[rows],
                    dst_ref=out_hbm.at[rows],
                    send_sem=yr_send.at[c],
                    recv_sem=yr_recv.at[c],
                    device_id=y_partner,
                    device_id_type=pl.DeviceIdType.MESH,
                )
                yr.start()
                yr_rdmas.append(yr)

        for i in range(H):
            yr_rdmas[i].wait_recv()
            zr_rdmas[i].wait_recv()
            yr_rdmas[i].wait_send()
            zr_rdmas[i].wait_send()
        for c in range(K):
            x_rdmas[c].wait_send()
        yd_rdmas[K - 2].wait_send()
        yd_rdmas[K - 1].wait_send()
        zd_rdmas[K - 2].wait_send()
        zd_rdmas[K - 1].wait_send()

    out, _recv = pl.pallas_call(
        body,
        out_shape=[
            jax.ShapeDtypeStruct((m, n), x.dtype),
            jax.ShapeDtypeStruct((Q, n), x.dtype),
        ],
        in_specs=[pl.BlockSpec(memory_space=pltpu.MemorySpace.HBM)],
        out_specs=[
            pl.BlockSpec(memory_space=pltpu.MemorySpace.HBM),
            pl.BlockSpec(memory_space=pltpu.MemorySpace.HBM),
        ],
        scratch_shapes=[
            pltpu.VMEM((2, rc, n), x.dtype),
            pltpu.VMEM((2, rc, n), x.dtype),
            pltpu.VMEM((2, rc, n), x.dtype),
            pltpu.SemaphoreType.DMA((K,)),
            pltpu.SemaphoreType.DMA((K,)),
            pltpu.SemaphoreType.DMA((K,)),
            pltpu.SemaphoreType.DMA((K,)),
            pltpu.SemaphoreType.DMA((K,)),
            pltpu.SemaphoreType.DMA((K,)),
            pltpu.SemaphoreType.DMA((H,)),
            pltpu.SemaphoreType.DMA((H,)),
            pltpu.SemaphoreType.DMA((H,)),
            pltpu.SemaphoreType.DMA((H,)),
            pltpu.SemaphoreType.DMA((2,)),
            pltpu.SemaphoreType.DMA((2,)),
            pltpu.SemaphoreType.DMA((2,)),
        ],
        compiler_params=pltpu.CompilerParams(collective_id=0),
    )(x)
    return out


# device time: 174371 ns/iter; 2.4843x vs baseline; 1.0303x over previous
import jax
import jax.numpy as jnp
from jax import lax
from jax.experimental import pallas as pl
from jax.experimental.pallas import tpu as pltpu

K = 16


def kernel(x):
    m, n = x.shape
    Q = m // 4
    rc = Q // K
    H = K // 2

    def body(x_hbm, out_hbm, recv_hbm, vx, vr, vs,
             x_send, x_recv, yd_send, yd_recv, zd_send, zd_recv,
             yr_send, yr_recv, zr_send, zr_recv,
             cpx_sem, cpr_sem, cpo_sem):
        my_x = lax.axis_index("x")
        my_y = lax.axis_index("y")
        my_z = lax.axis_index("z")
        p = my_y % 2
        r = my_z % 2
        x_partner = (1 - my_x, my_y, my_z)
        y_partner = (my_x, my_y ^ 1, my_z)
        z_partner = (my_x, my_y, my_z ^ 1)

        q_me = 2 * p + r
        q_yp = 2 * (1 - p) + r
        q_zp = 2 * p + (1 - r)

        barrier_sem = pltpu.get_barrier_semaphore()
        for nbr in (x_partner, y_partner, z_partner):
            pl.semaphore_signal(
                barrier_sem, inc=1,
                device_id=nbr, device_id_type=pl.DeviceIdType.MESH,
            )
        pl.semaphore_wait(barrier_sem, 3)

        x_rdmas = []
        for c in range(K):
            rdma = pltpu.make_async_remote_copy(
                src_ref=x_hbm.at[pl.ds(q_me * Q + c * rc, rc)],
                dst_ref=recv_hbm.at[pl.ds(c * rc, rc)],
                send_sem=x_send.at[c],
                recv_sem=x_recv.at[c],
                device_id=x_partner,
                device_id_type=pl.DeviceIdType.MESH,
            )
            rdma.start()
            x_rdmas.append(rdma)

        yd_rdmas = []
        zd_rdmas = []
        for c in range(K):
            slot = c % 2
            if c >= 2:
                yd_rdmas[c - 2].wait_send()
                zd_rdmas[c - 2].wait_send()
            rows = pl.ds(q_me * Q + c * rc, rc)
            cp_x = pltpu.make_async_copy(x_hbm.at[rows], vx.at[slot], cpx_sem.at[slot])
            cp_x.start()
            x_rdmas[c].wait_recv()
            cp_r = pltpu.make_async_copy(
                recv_hbm.at[pl.ds(c * rc, rc)], vr.at[slot], cpr_sem.at[slot]
            )
            cp_r.start()
            cp_x.wait()
            cp_r.wait()
            vs[slot] = vx[slot] + vr[slot]
            cp_o = pltpu.make_async_copy(vs.at[slot], out_hbm.at[rows], cpo_sem.at[slot])
            cp_o.start()
            yd = pltpu.make_async_remote_copy(
                src_ref=vs.at[slot],
                dst_ref=out_hbm.at[rows],
                send_sem=yd_send.at[c],
                recv_sem=yd_recv.at[c],
                device_id=y_partner,
                device_id_type=pl.DeviceIdType.MESH,
            )
            yd.start()
            yd_rdmas.append(yd)
            zd = pltpu.make_async_remote_copy(
                src_ref=vs.at[slot],
                dst_ref=out_hbm.at[rows],
                send_sem=zd_send.at[c],
                recv_sem=zd_recv.at[c],
                device_id=z_partner,
                device_id_type=pl.DeviceIdType.MESH,
            )
            zd.start()
            zd_rdmas.append(zd)
            cp_o.wait()

        yr_rdmas = []
        zr_rdmas = []
        for c in range(K):
            yd_rdmas[c].wait_recv()
            if c >= H:
                rows = pl.ds(q_yp * Q + c * rc, rc)
                zr = pltpu.make_async_remote_copy(
                    src_ref=out_hbm.at[rows],
                    dst_ref=out_hbm.at[rows],
                    send_sem=zr_send.at[c - H],
                    recv_sem=zr_recv.at[c - H],
                    device_id=z_partner,
                    device_id_type=pl.DeviceIdType.MESH,
                )
                zr.start()
                zr_rdmas.append(zr)
            zd_rdmas[c].wait_recv()
            if c < H:
                rows = pl.ds(q_zp * Q + c * rc, rc)
                yr = pltpu.make_async_remote_copy(
                    src_ref=out_hbm.at[rows],
                    dst_ref=out_hbm.at[rows],
                    send_sem=yr_send.at[c],
                    recv_sem=yr_recv.at[c],
                    device_id=y_partner,
                    device_id_type=pl.DeviceIdType.MESH,
                )
                yr.start()
                yr_rdmas.append(yr)

        for i in range(H):
            yr_rdmas[i].wait_recv()
            zr_rdmas[i].wait_recv()
            yr_rdmas[i].wait_send()
            zr_rdmas[i].wait_send()
        for c in range(K):
            x_rdmas[c].wait_send()
        yd_rdmas[K - 2].wait_send()
        yd_rdmas[K - 1].wait_send()
        zd_rdmas[K - 2].wait_send()
        zd_rdmas[K - 1].wait_send()

    out, _recv = pl.pallas_call(
        body,
        out_shape=[
            jax.ShapeDtypeStruct((m, n), x.dtype),
            jax.ShapeDtypeStruct((Q, n), x.dtype),
        ],
        in_specs=[pl.BlockSpec(memory_space=pltpu.MemorySpace.HBM)],
        out_specs=[
            pl.BlockSpec(memory_space=pltpu.MemorySpace.HBM),
            pl.BlockSpec(memory_space=pltpu.MemorySpace.HBM),
        ],
        scratch_shapes=[
            pltpu.VMEM((2, rc, n), x.dtype),
            pltpu.VMEM((2, rc, n), x.dtype),
            pltpu.VMEM((2, rc, n), x.dtype),
            pltpu.SemaphoreType.DMA((K,)),
            pltpu.SemaphoreType.DMA((K,)),
            pltpu.SemaphoreType.DMA((K,)),
            pltpu.SemaphoreType.DMA((K,)),
            pltpu.SemaphoreType.DMA((K,)),
            pltpu.SemaphoreType.DMA((K,)),
            pltpu.SemaphoreType.DMA((H,)),
            pltpu.SemaphoreType.DMA((H,)),
            pltpu.SemaphoreType.DMA((H,)),
            pltpu.SemaphoreType.DMA((H,)),
            pltpu.SemaphoreType.DMA((2,)),
            pltpu.SemaphoreType.DMA((2,)),
            pltpu.SemaphoreType.DMA((2,)),
        ],
        compiler_params=pltpu.CompilerParams(collective_id=0),
    )(x)
    return out


# device time: 173086 ns/iter; 2.5027x vs baseline; 1.0074x over previous
import jax
import jax.numpy as jnp
from jax import lax
from jax.experimental import pallas as pl
from jax.experimental.pallas import tpu as pltpu

K = 16
H = K // 2


def kernel(x):
    m, n = x.shape
    Q = m // 4
    rc = Q // K
    MESH = pl.DeviceIdType.MESH

    def body(x_hbm, out_hbm, vq, vrecv, vs,
             x_send, x_recv, yd_send, yd_recv, zd_send, zd_recv,
             yr_send, yr_recv, zr_send, zr_recv,
             cpq_sem, cpo_sem):
        my_x = lax.axis_index("x")
        my_y = lax.axis_index("y")
        my_z = lax.axis_index("z")
        p = my_y % 2
        r = my_z % 2
        x_partner = (1 - my_x, my_y, my_z)
        y_partner = (my_x, my_y ^ 1, my_z)
        z_partner = (my_x, my_y, my_z ^ 1)

        q_me = 2 * p + r
        q_yp = 2 * (1 - p) + r
        q_zp = 2 * p + (1 - r)

        cp_q = pltpu.make_async_copy(
            x_hbm.at[pl.ds(q_me * Q, Q)], vq, cpq_sem)
        cp_q.start()

        barrier_sem = pltpu.get_barrier_semaphore()
        for nbr in (x_partner, y_partner, z_partner):
            pl.semaphore_signal(
                barrier_sem, inc=1, device_id=nbr, device_id_type=MESH,
            )
        pl.semaphore_wait(barrier_sem, 3)

        x_rdmas = []
        for c in range(K):
            rdma = pltpu.make_async_remote_copy(
                src_ref=x_hbm.at[pl.ds(q_me * Q + c * rc, rc)],
                dst_ref=vrecv.at[pl.ds(c * rc, rc)],
                send_sem=x_send.at[c],
                recv_sem=x_recv.at[c],
                device_id=x_partner,
                device_id_type=MESH,
            )
            rdma.start()
            x_rdmas.append(rdma)

        cp_q.wait()

        yd_rdmas = []
        zd_rdmas = []
        cp_os = []
        for c in range(K):
            slot = c % 2
            if c >= 2:
                yd_rdmas[c - 2].wait_send()
                zd_rdmas[c - 2].wait_send()
                cp_os[c - 2].wait()
            rows = pl.ds(q_me * Q + c * rc, rc)
            x_rdmas[c].wait_recv()
            vs[slot] = (vq[pl.ds(c * rc, rc)] +
                        vrecv[pl.ds(c * rc, rc)])
            cp_o = pltpu.make_async_copy(
                vs.at[slot], out_hbm.at[rows], cpo_sem.at[slot])
            cp_o.start()
            cp_os.append(cp_o)
            yd = pltpu.make_async_remote_copy(
                src_ref=vs.at[slot], dst_ref=out_hbm.at[rows],
                send_sem=yd_send.at[c], recv_sem=yd_recv.at[c],
                device_id=y_partner, device_id_type=MESH,
            )
            yd.start()
            yd_rdmas.append(yd)
            zd = pltpu.make_async_remote_copy(
                src_ref=vs.at[slot], dst_ref=out_hbm.at[rows],
                send_sem=zd_send.at[c], recv_sem=zd_recv.at[c],
                device_id=z_partner, device_id_type=MESH,
            )
            zd.start()
            zd_rdmas.append(zd)

        yr_rdmas = []
        zr_rdmas = []
        for c in range(K):
            yd_rdmas[c].wait_recv()
            if c >= H:
                rws = pl.ds(q_yp * Q + c * rc, rc)
                zr = pltpu.make_async_remote_copy(
                    src_ref=out_hbm.at[rws], dst_ref=out_hbm.at[rws],
                    send_sem=zr_send.at[c - H], recv_sem=zr_recv.at[c - H],
                    device_id=z_partner, device_id_type=MESH,
                )
                zr.start()
                zr_rdmas.append(zr)
            zd_rdmas[c].wait_recv()
            if c < H:
                rws = pl.ds(q_zp * Q + c * rc, rc)
                yr = pltpu.make_async_remote_copy(
                    src_ref=out_hbm.at[rws], dst_ref=out_hbm.at[rws],
                    send_sem=yr_send.at[c], recv_sem=yr_recv.at[c],
                    device_id=y_partner, device_id_type=MESH,
                )
                yr.start()
                yr_rdmas.append(yr)

        for i in range(H):
            yr_rdmas[i].wait_recv()
            zr_rdmas[i].wait_recv()
            yr_rdmas[i].wait_send()
            zr_rdmas[i].wait_send()
        for c in range(K):
            x_rdmas[c].wait_send()
        for c in (K - 2, K - 1):
            yd_rdmas[c].wait_send()
            zd_rdmas[c].wait_send()
            cp_os[c].wait()

    out = pl.pallas_call(
        body,
        out_shape=jax.ShapeDtypeStruct((m, n), x.dtype),
        in_specs=[pl.BlockSpec(memory_space=pltpu.MemorySpace.HBM)],
        out_specs=pl.BlockSpec(memory_space=pltpu.MemorySpace.HBM),
        scratch_shapes=[
            pltpu.VMEM((Q, n), x.dtype),
            pltpu.VMEM((Q, n), x.dtype),
            pltpu.VMEM((2, rc, n), x.dtype),
            pltpu.SemaphoreType.DMA((K,)),
            pltpu.SemaphoreType.DMA((K,)),
            pltpu.SemaphoreType.DMA((K,)),
            pltpu.SemaphoreType.DMA((K,)),
            pltpu.SemaphoreType.DMA((K,)),
            pltpu.SemaphoreType.DMA((K,)),
            pltpu.SemaphoreType.DMA((H,)),
            pltpu.SemaphoreType.DMA((H,)),
            pltpu.SemaphoreType.DMA((H,)),
            pltpu.SemaphoreType.DMA((H,)),
            pltpu.SemaphoreType.DMA,
            pltpu.SemaphoreType.DMA((2,)),
        ],
        compiler_params=pltpu.CompilerParams(collective_id=0),
    )(x)
    return out
